# baseline (speedup 1.0000x reference)
import functools

import jax
import jax.numpy as jnp
from jax import lax
from jax.experimental import pallas as pl
from jax.experimental.pallas import tpu as pltpu
from jax.experimental.pallas import tpu_sc as plsc

_NBINS = 2048
_ZERO_POINT = 0.0004
_L = 16
_UNROLL = 8


@functools.cache
def _make_sc_kernel(rows: int, cols: int):
    assert cols == _NBINS
    info = plsc.get_sparse_core_info()
    nc, ns = info.num_cores, info.num_subcores
    nw = nc * ns
    assert rows % nw == 0
    rows_per_w = rows // nw
    assert rows_per_w % 2 == 0

    mesh = plsc.VectorSubcoreMesh(core_axis_name="c", subcore_axis_name="s")

    @functools.partial(
        pl.kernel,
        out_type=jax.ShapeDtypeStruct((rows, cols), jnp.float32),
        mesh=mesh,
        compiler_params=pltpu.CompilerParams(needs_layout_passes=False),
        scratch_types=[
            pltpu.VMEM((cols,), jnp.float32),
            pltpu.VMEM((cols,), jnp.float32),
            pltpu.VMEM((cols,), jnp.float32),
            pltpu.VMEM((cols,), jnp.float32),
            pltpu.VMEM((cols,), jnp.float32),
            pltpu.VMEM((cols,), jnp.float32),
            pltpu.VMEM((cols,), jnp.float32),
            pltpu.VMEM((cols,), jnp.float32),
            pltpu.VMEM((cols,), jnp.float32),
            pltpu.VMEM((cols,), jnp.float32),
            pltpu.VMEM((_L,), jnp.float32),
            pltpu.SemaphoreType.DMA,
            pltpu.SemaphoreType.DMA,
            pltpu.SemaphoreType.DMA,
            pltpu.SemaphoreType.DMA,
            pltpu.SemaphoreType.DMA,
            pltpu.SemaphoreType.DMA,
            pltpu.SemaphoreType.DMA,
        ],
    )
    def sc_kernel(mu_h, scale_h, zeros_h, eps_h, x_h, out_h,
                  mu_v, scale_v, eps_b0, eps_b1, x_b0, x_b1, zs_v, cnt_v,
                  out_b0, out_b1, rsum_v, se0, se1, sx0, sx1, so0, so1, sz):
        wid = lax.axis_index("s") * nc + lax.axis_index("c")
        row0 = wid * rows_per_w
        eps_bufs = (eps_b0, eps_b1)
        x_bufs = (x_b0, x_b1)
        out_bufs = (out_b0, out_b1)
        se = (se0, se1)
        sx = (sx0, sx1)
        so = (so0, so1)
        pltpu.sync_copy(mu_h, mu_v)
        pltpu.sync_copy(scale_h, scale_v)

        zeros = jnp.zeros((_L,), jnp.float32)
        ones = jnp.ones((_L,), jnp.float32)

        pltpu.async_copy(zeros_h, cnt_v, sz)

        pltpu.async_copy(eps_h.at[row0], eps_bufs[0], se[0])
        pltpu.async_copy(x_h.at[row0], x_bufs[0], sx[0])

        def do_row(r, b):
            row = row0 + r
            eps_v = eps_bufs[b]
            x_v = x_bufs[b]
            out_v = out_bufs[b]

            @pl.when(r + 1 < rows_per_w)
            def _():
                pltpu.async_copy(eps_h.at[row + 1], eps_bufs[1 - b],
                                 se[1 - b])
                pltpu.async_copy(x_h.at[row + 1], x_bufs[1 - b],
                                 sx[1 - b])

            pltpu.make_async_copy(eps_h.at[row], eps_v, se[b]).wait()
            pltpu.make_async_copy(x_h.at[row], x_v, sx[b]).wait()

            @plsc.parallel_loop(
                0, cols, _L, unroll=_UNROLL,
                carry=(jnp.full((_L,), jnp.inf, jnp.float32),
                       jnp.full((_L,), -jnp.inf, jnp.float32)))
            def minmax(i, carry):
                mn, mx = carry
                sl = pl.ds(i, _L)
                p = mu_v[sl] + scale_v[sl] * eps_v[sl]
                zs_v[sl] = p
                return jnp.minimum(mn, p), jnp.maximum(mx, p)

            mn16, mx16 = minmax
            vmin16 = jnp.full((_L,), jnp.min(mn16), jnp.float32)
            vmax16 = jnp.full((_L,), jnp.max(mx16), jnp.float32)
            width16 = (vmax16 - vmin16) * jnp.float32(1.0 / _NBINS)
            safe16 = jnp.where(width16 == 0.0, jnp.float32(1.0), width16)

            pltpu.make_async_copy(zeros_h, cnt_v, sz).wait()

            @plsc.parallel_loop(0, cols, _L, unroll=_UNROLL)
            def _(i):
                sl = pl.ds(i, _L)
                q = (zs_v[sl] - vmin16) / safe16
                idx = jnp.clip(q.astype(jnp.int32), 0, _NBINS - 1)
                plsc.addupdate_scatter(cnt_v, [idx], ones)

            g16 = jnp.full((_L,), jnp.float32(32.0))

            @plsc.parallel_loop(0, cols, _L, unroll=_UNROLL, carry=zeros)
            def ssum(i, acc):
                sl = pl.ds(i, _L)
                u = jnp.exp(cnt_v[sl] - g16)
                zs_v[sl] = u
                return acc + u

            s16 = jnp.full((_L,), jnp.sum(ssum), jnp.float32)
            rsum_v[...] = ones / s16

            @pl.when(jnp.any(s16 >= jnp.float32(3.0e38)))
            def _():
                @plsc.parallel_loop(0, cols, _L, unroll=_UNROLL,
                                    carry=zeros)
                def cmax(i, mx):
                    return jnp.maximum(mx, cnt_v[pl.ds(i, _L)])

                cm16 = jnp.full((_L,), jnp.max(cmax), jnp.float32)

                @plsc.parallel_loop(0, cols, _L, unroll=_UNROLL,
                                    carry=zeros)
                def ssum2(i, acc):
                    sl = pl.ds(i, _L)
                    u = jnp.exp(cnt_v[sl] - cm16)
                    zs_v[sl] = u
                    return acc + u

                rsum_v[...] = ones / jnp.full((_L,), jnp.sum(ssum2),
                                              jnp.float32)

            r16 = rsum_v[...]

            pltpu.async_copy(zeros_h, cnt_v, sz)

            @pl.when(r >= 2)
            def _():
                pltpu.make_async_copy(out_v, out_h.at[row - 2], so[b]).wait()

            @plsc.parallel_loop(0, cols, _L, unroll=_UNROLL)
            def _(i):
                sl = pl.ds(i, _L)
                probs = zs_v[sl] * r16
                probs = jnp.where(probs < _ZERO_POINT,
                                  jnp.float32(0.0), probs)
                out_v[sl] = x_v[sl] * probs

            pltpu.async_copy(out_v, out_h.at[row], so[b])

        def pair_body(rb, _):
            do_row(rb * 2, 0)
            do_row(rb * 2 + 1, 1)
            return 0

        lax.fori_loop(0, rows_per_w // 2, pair_body, 0)

        pltpu.make_async_copy(zeros_h, cnt_v, sz).wait()

        last = row0 + rows_per_w - 2
        pltpu.make_async_copy(out_bufs[0], out_h.at[last], so[0]).wait()
        pltpu.make_async_copy(out_bufs[1], out_h.at[last + 1], so[1]).wait()

    return sc_kernel


def kernel(z, x, epsilon):
    mu = jnp.mean(z, axis=0)
    var = jnp.var(z, axis=0)
    scale = jnp.exp(0.5 * var)
    rows, cols = epsilon.shape
    zeros_hbm = jnp.zeros((cols,), jnp.float32)
    return _make_sc_kernel(rows, cols)(mu, scale, zeros_hbm, epsilon, x)


# speedup vs baseline: 2.0558x; 2.0558x over previous
import functools

import jax
import jax.numpy as jnp
from jax import lax
from jax.experimental import pallas as pl
from jax.experimental.pallas import tpu as pltpu
from jax.experimental.pallas import tpu_sc as plsc

_NBINS = 2048
_ZERO_POINT = 0.0004
_L = 16
_UNROLL = 8


@functools.cache
def _make_sc_kernel(rows: int, cols: int):
    assert cols == _NBINS
    info = plsc.get_sparse_core_info()
    nc, ns = info.num_cores, info.num_subcores
    nw = nc * ns
    assert rows % nw == 0
    rows_per_w = rows // nw
    assert rows_per_w % 2 == 0

    mesh = plsc.VectorSubcoreMesh(core_axis_name="c", subcore_axis_name="s")

    @functools.partial(
        pl.kernel,
        out_type=jax.ShapeDtypeStruct((rows, cols), jnp.float32),
        mesh=mesh,
        compiler_params=pltpu.CompilerParams(needs_layout_passes=False),
        scratch_types=[
            pltpu.VMEM((cols,), jnp.float32),
            pltpu.VMEM((cols,), jnp.float32),
            pltpu.VMEM((cols,), jnp.float32),
            pltpu.VMEM((cols,), jnp.float32),
            pltpu.VMEM((cols,), jnp.float32),
            pltpu.VMEM((cols,), jnp.float32),
            pltpu.VMEM((cols,), jnp.float32),
            pltpu.VMEM((cols,), jnp.float32),
            pltpu.VMEM((cols,), jnp.float32),
            pltpu.VMEM((cols,), jnp.float32),
            pltpu.VMEM((_L,), jnp.float32),
            pltpu.SemaphoreType.DMA,
            pltpu.SemaphoreType.DMA,
            pltpu.SemaphoreType.DMA,
            pltpu.SemaphoreType.DMA,
            pltpu.SemaphoreType.DMA,
            pltpu.SemaphoreType.DMA,
        ],
    )
    def sc_kernel(mu_h, scale_h, eps_h, x_h, out_h,
                  mu_v, scale_v, eps_b0, eps_b1, x_b0, x_b1, zs_v, cnt_v,
                  out_b0, out_b1, rsum_v, se0, se1, sx0, sx1, so0, so1):
        wid = lax.axis_index("s") * nc + lax.axis_index("c")
        row0 = wid * rows_per_w
        eps_bufs = (eps_b0, eps_b1)
        x_bufs = (x_b0, x_b1)
        out_bufs = (out_b0, out_b1)
        se = (se0, se1)
        sx = (sx0, sx1)
        so = (so0, so1)
        pltpu.sync_copy(mu_h, mu_v)
        pltpu.sync_copy(scale_h, scale_v)

        zeros = jnp.zeros((_L,), jnp.float32)
        ones = jnp.ones((_L,), jnp.float32)

        @plsc.parallel_loop(0, cols, _L, unroll=_UNROLL)
        def _(i):
            cnt_v[pl.ds(i, _L)] = zeros

        pltpu.async_copy(eps_h.at[row0], eps_bufs[0], se[0])
        pltpu.async_copy(x_h.at[row0], x_bufs[0], sx[0])

        def do_row(r, b):
            row = row0 + r
            eps_v = eps_bufs[b]
            x_v = x_bufs[b]
            out_v = out_bufs[b]

            @pl.when(r + 1 < rows_per_w)
            def _():
                pltpu.async_copy(eps_h.at[row + 1], eps_bufs[1 - b],
                                 se[1 - b])
                pltpu.async_copy(x_h.at[row + 1], x_bufs[1 - b],
                                 sx[1 - b])

            pltpu.make_async_copy(eps_h.at[row], eps_v, se[b]).wait()
            pltpu.make_async_copy(x_h.at[row], x_v, sx[b]).wait()

            @plsc.parallel_loop(
                0, cols, _L, unroll=_UNROLL,
                carry=(jnp.full((_L,), jnp.inf, jnp.float32),
                       jnp.full((_L,), -jnp.inf, jnp.float32)))
            def minmax(i, carry):
                mn, mx = carry
                sl = pl.ds(i, _L)
                p = mu_v[sl] + scale_v[sl] * eps_v[sl]
                zs_v[sl] = p
                return jnp.minimum(mn, p), jnp.maximum(mx, p)

            mn16, mx16 = minmax
            vmin16 = jnp.full((_L,), jnp.min(mn16), jnp.float32)
            vmax16 = jnp.full((_L,), jnp.max(mx16), jnp.float32)
            width16 = (vmax16 - vmin16) * jnp.float32(1.0 / _NBINS)
            safe16 = jnp.where(width16 == 0.0, jnp.float32(1.0), width16)

            @plsc.parallel_loop(0, cols, _L, unroll=_UNROLL)
            def _(i):
                sl = pl.ds(i, _L)
                q = (zs_v[sl] - vmin16) / safe16
                idx = jnp.clip(q.astype(jnp.int32), 0, _NBINS - 1)
                plsc.addupdate_scatter(cnt_v, [idx], ones)

            g16 = jnp.full((_L,), jnp.float32(32.0))

            @plsc.parallel_loop(0, cols, _L, unroll=_UNROLL, carry=zeros)
            def ssum(i, acc):
                sl = pl.ds(i, _L)
                u = jnp.exp(cnt_v[sl] - g16)
                zs_v[sl] = u
                return acc + u

            s16 = jnp.full((_L,), jnp.sum(ssum), jnp.float32)
            rsum_v[...] = ones / s16

            @pl.when(jnp.any(s16 >= jnp.float32(3.0e38)))
            def _():
                @plsc.parallel_loop(0, cols, _L, unroll=_UNROLL,
                                    carry=zeros)
                def cmax(i, mx):
                    return jnp.maximum(mx, cnt_v[pl.ds(i, _L)])

                cm16 = jnp.full((_L,), jnp.max(cmax), jnp.float32)

                @plsc.parallel_loop(0, cols, _L, unroll=_UNROLL,
                                    carry=zeros)
                def ssum2(i, acc):
                    sl = pl.ds(i, _L)
                    u = jnp.exp(cnt_v[sl] - cm16)
                    zs_v[sl] = u
                    return acc + u

                rsum_v[...] = ones / jnp.full((_L,), jnp.sum(ssum2),
                                              jnp.float32)

            r16 = rsum_v[...]

            @pl.when(r >= 2)
            def _():
                pltpu.make_async_copy(out_v, out_h.at[row - 2], so[b]).wait()

            @plsc.parallel_loop(0, cols, _L, unroll=_UNROLL)
            def _(i):
                sl = pl.ds(i, _L)
                probs = zs_v[sl] * r16
                probs = jnp.where(probs < _ZERO_POINT,
                                  jnp.float32(0.0), probs)
                out_v[sl] = x_v[sl] * probs
                cnt_v[sl] = zeros

            pltpu.async_copy(out_v, out_h.at[row], so[b])

        def pair_body(rb, _):
            do_row(rb * 2, 0)
            do_row(rb * 2 + 1, 1)
            return 0

        lax.fori_loop(0, rows_per_w // 2, pair_body, 0)

        last = row0 + rows_per_w - 2
        pltpu.make_async_copy(out_bufs[0], out_h.at[last], so[0]).wait()
        pltpu.make_async_copy(out_bufs[1], out_h.at[last + 1], so[1]).wait()

    return sc_kernel


def kernel(z, x, epsilon):
    mu = jnp.mean(z, axis=0)
    var = jnp.var(z, axis=0)
    scale = jnp.exp(0.5 * var)
    rows, cols = epsilon.shape
    return _make_sc_kernel(rows, cols)(mu, scale, epsilon, x)
